# baseline (speedup 1.0000x reference)
import jax
import jax.numpy as jnp
from jax import lax
from jax.experimental import pallas as pl
from jax.experimental.pallas import tpu as pltpu
from jax.experimental.pallas import tpu_sc as plsc

_CONTEXT = 8192
_DIM = 64
_NUM_CORES = 2
_NUM_SUBCORES = 16
_NUM_WORKERS = _NUM_CORES * _NUM_SUBCORES
_ROWS_PER_WORKER = _CONTEXT // _NUM_WORKERS
_LANES = 16
_CHUNKS_PER_ROW = _DIM // _LANES
_ROWS_PER_STEP = 16


def _emb_body(x_hbm, table_hbm, pos_hbm, out_hbm, idx_v, rows_v, pos_v, sem):
    wid = lax.axis_index("s") * _NUM_CORES + lax.axis_index("c")
    base = wid * _ROWS_PER_WORKER

    pltpu.sync_copy(x_hbm.at[pl.ds(base, _ROWS_PER_WORKER)], idx_v)
    gather = pltpu.async_copy(table_hbm.at[idx_v], rows_v, sem)
    pltpu.sync_copy(pos_hbm.at[pl.ds(base, _ROWS_PER_WORKER)], pos_v)
    gather.wait()

    def add_step(i, carry):
        r0 = i * _ROWS_PER_STEP
        for r in range(_ROWS_PER_STEP):
            for c in range(_CHUNKS_PER_ROW):
                sl = pl.ds(c * _LANES, _LANES)
                rows_v[r0 + r, sl] = rows_v[r0 + r, sl] + pos_v[r0 + r, sl]
        return carry

    lax.fori_loop(0, _ROWS_PER_WORKER // _ROWS_PER_STEP, add_step, 0)

    pltpu.sync_copy(rows_v, out_hbm.at[pl.ds(base, _ROWS_PER_WORKER)])


def kernel(x, table, pos_enc):
    mesh = plsc.VectorSubcoreMesh(core_axis_name="c", subcore_axis_name="s")
    f = pl.kernel(
        _emb_body,
        mesh=mesh,
        compiler_params=pltpu.CompilerParams(use_tc_tiling_on_sc=False),
        out_type=jax.ShapeDtypeStruct((_CONTEXT, _DIM), jnp.float32),
        scratch_types=[
            pltpu.VMEM((_ROWS_PER_WORKER,), jnp.int32),
            pltpu.VMEM((_ROWS_PER_WORKER, _DIM), jnp.float32),
            pltpu.VMEM((_ROWS_PER_WORKER, _DIM), jnp.float32),
            pltpu.SemaphoreType.DMA,
        ],
    )
    return f(x, table, pos_enc)
